# baseline (speedup 1.0000x reference)
import functools

import jax
import jax.numpy as jnp
from jax import lax
from jax.experimental import pallas as pl
from jax.experimental.pallas import tpu as pltpu
from jax.experimental.pallas import tpu_sc as plsc

_HID = 64
_PAD = 128
_LANES = 16
_NBUF = 2


@functools.lru_cache(maxsize=None)
def _build(n_rows: int, seq: int, vocab: int):
  info = plsc.get_sparse_core_info()
  nw = info.num_cores * info.num_subcores
  rows_per_w = n_rows // nw
  chunk = seq
  n_chunks = rows_per_w // chunk
  assert n_chunks % _NBUF == 0
  mesh = plsc.VectorSubcoreMesh(core_axis_name="c", subcore_axis_name="s")

  @functools.partial(
      pl.kernel,
      mesh=mesh,
      out_type=jax.ShapeDtypeStruct((n_rows // seq, seq, _HID), jnp.float32),
      scratch_types=[
          pltpu.VMEM((seq, _PAD), jnp.float32),
          pltpu.VMEM((n_chunks, chunk), jnp.int32),
          pltpu.VMEM((_NBUF, chunk, _PAD), jnp.float32),
          pltpu.VMEM((_NBUF, chunk, _HID), jnp.float32),
      ] + [pltpu.SemaphoreType.DMA] * (2 * _NBUF),
  )
  def emb(idx_hbm, tok_hbm, pos_hbm, out_hbm, pos_v, idx_v, gat, obuf, *sems):
    sem_in = sems[:_NBUF]
    sem_out = sems[_NBUF:]
    wid = lax.axis_index("s") * info.num_cores + lax.axis_index("c")
    w_batch = wid * n_chunks
    pltpu.sync_copy(pos_hbm, pos_v)
    pltpu.sync_copy(idx_hbm.at[wid], idx_v)

    def gather(c, slot):
      return pltpu.make_async_copy(
          tok_hbm.at[idx_v.at[c]], gat.at[slot], sem_in[slot])

    def scatter(c, slot):
      return pltpu.make_async_copy(
          obuf.at[slot], out_hbm.at[w_batch + c], sem_out[slot])

    gather(0, 0).start()

    @pl.loop(0, n_chunks, step=_NBUF)
    def _group(g):
      for b in range(_NBUF):
        c = g + b

        @pl.when(c + 1 < n_chunks)
        def _prefetch():
          gather(c + 1, 1 - b).start()

        gather(c, b).wait()

        @pl.when(c >= _NBUF)
        def _retire():
          scatter(c - _NBUF, b).wait()

        @pl.loop(0, chunk, unroll=4)
        def _row(s):
          for h in range(_HID // _LANES):
            sl = pl.ds(h * _LANES, _LANES)
            obuf[b, s, sl] = gat[b, s, sl] + pos_v[s, sl]

        scatter(c, b).start()

    for b in range(_NBUF):
      scatter(n_chunks - _NBUF + b, b).wait()

  return emb


def kernel(x, tok_table, pos_table):
  b, s = x.shape
  vocab, hid = tok_table.shape
  info = plsc.get_sparse_core_info()
  nw = info.num_cores * info.num_subcores
  n_rows = b * s
  chunk = s
  xf = x.reshape(nw, n_rows // nw // chunk, chunk).astype(jnp.int32)
  tok_pad = jnp.pad(tok_table, ((0, 0), (0, _PAD - hid)))
  pos_pad = jnp.pad(pos_table, ((0, 0), (0, _PAD - hid)))
  return _build(n_rows, s, vocab)(xf, tok_pad, pos_pad)
